# baseline (speedup 1.0000x reference)
import functools

import jax
import jax.numpy as jnp
from jax import lax
from jax.experimental import pallas as pl
from jax.experimental.pallas import tpu as pltpu
from jax.experimental.pallas import tpu_sc as plsc

_N = 10000
_F = 128
_NC = 2
_NS = 16
_NW = _NC * _NS
_CK = 64
_NCH = 160
_NCHH = 80
_ACC = 10240
_RPT = _ACC // _NS
_EPAD = _NW * _NCH * _CK

_mesh = plsc.VectorSubcoreMesh(core_axis_name="c", subcore_axis_name="s")


_CW = 128


@functools.partial(
    pl.kernel,
    out_type=jax.ShapeDtypeStruct((_NC, _ACC, _CW), jnp.float32),
    mesh=_mesh,
    scratch_types=[
        pltpu.VMEM((_NCH, _CK), jnp.int32),
        pltpu.VMEM((_CK, _CW), jnp.float32),
        pltpu.VMEM_SHARED((_ACC, _CW), jnp.float32),
    ],
)
def _sc_cnt(dsts, zeros_f, ones_f, cnt_out, dst_v, buf_v, cacc):
    c = lax.axis_index("c")
    s = lax.axis_index("s")
    w = c * _NS + s
    pltpu.sync_copy(dsts.at[w], dst_v)
    pltpu.sync_copy(zeros_f, buf_v)

    @pl.loop(0, _ACC // (_NS * _CK))
    def _(zb):
        base = (s * (_ACC // (_NS * _CK)) + zb) * _CK
        pltpu.sync_copy(buf_v, cacc.at[pl.ds(base, _CK)])

    pltpu.sync_copy(ones_f, buf_v)
    plsc.subcore_barrier()

    @pl.loop(0, _NCH)
    def _(j):
        pltpu.sync_copy(buf_v, cacc.at[dst_v.at[j]], add=True)

    plsc.subcore_barrier()
    pltpu.sync_copy(cacc.at[pl.ds(s * _RPT, _RPT)],
                    cnt_out.at[c, pl.ds(s * _RPT, _RPT)])


@functools.partial(
    pl.kernel,
    out_type=jax.ShapeDtypeStruct((_NC, _ACC, _F), jnp.float32),
    mesh=_mesh,
    scratch_types=[
        pltpu.VMEM((_NCHH, _CK), jnp.int32),
        pltpu.VMEM((_NCHH, _CK), jnp.int32),
        pltpu.VMEM((_CK, _F), jnp.float32),
        pltpu.VMEM((_CK, _F), jnp.float32),
        pltpu.VMEM_SHARED((_ACC, _F), jnp.float32),
        pltpu.SemaphoreType.DMA,
        pltpu.SemaphoreType.DMA,
    ],
)
def _sc_segsum(h, srcs, dsts, zeros_f, seg_out, src_v, dst_v, rows_a, rows_b,
               acc, sema, semb):
    c = lax.axis_index("c")
    s = lax.axis_index("s")
    w = c * _NS + s
    pltpu.sync_copy(zeros_f, rows_a)

    @pl.loop(0, _ACC // (_NS * _CK))
    def _(zb):
        base = (s * (_ACC // (_NS * _CK)) + zb) * _CK
        pltpu.sync_copy(rows_a, acc.at[pl.ds(base, _CK)])

    plsc.subcore_barrier()

    for half in range(_NCH // _NCHH):
        pltpu.sync_copy(srcs.at[w, pl.ds(half * _NCHH, _NCHH)], src_v)
        pltpu.sync_copy(dsts.at[w, pl.ds(half * _NCHH, _NCHH)], dst_v)

        @pl.loop(0, _NCHH, step=2)
        def _(j):
            da = pltpu.async_copy(h.at[src_v.at[j]], rows_a, sema)
            db = pltpu.async_copy(h.at[src_v.at[j + 1]], rows_b, semb)
            da.wait()
            pltpu.sync_copy(rows_a, acc.at[dst_v.at[j]], add=True)
            db.wait()
            pltpu.sync_copy(rows_b, acc.at[dst_v.at[j + 1]], add=True)

    plsc.subcore_barrier()
    pltpu.sync_copy(acc.at[pl.ds(s * _RPT, _RPT)],
                    seg_out.at[c, pl.ds(s * _RPT, _RPT)])


_BLK = 2000


def _mm_t(a, w):
    return lax.dot_general(a, w, (((1,), (1,)), ((), ())),
                           preferred_element_type=jnp.float32)


def _root_body(h_ref, w_ref, b_ref, o_ref):
    o_ref[...] = _mm_t(h_ref[...], w_ref[...]) + b_ref[...]


def _tc_root(h, Wr, bl2d):
    return pl.pallas_call(
        _root_body,
        grid=(_N // _BLK,),
        in_specs=[
            pl.BlockSpec((_BLK, _F), lambda i: (i, 0)),
            pl.BlockSpec((_F, _F), lambda i: (0, 0)),
            pl.BlockSpec((1, _F), lambda i: (0, 0)),
        ],
        out_specs=pl.BlockSpec((_BLK, _F), lambda i: (i, 0)),
        out_shape=jax.ShapeDtypeStruct((_N, _F), jnp.float32),
    )(h, Wr, bl2d)


def _combine_body(s_ref, c_ref, r_ref, w_ref, o_ref):
    cl = c_ref[...]
    cnt = jnp.maximum(cl[0, :, 0:1] + cl[1, :, 0:1], 1.0)
    agg = (s_ref[0] + s_ref[1]) / cnt
    o_ref[...] = jnp.maximum(_mm_t(agg, w_ref[...]) + r_ref[...], 0.0)


def _tc_combine(seg, cnt, r, Wl):
    return pl.pallas_call(
        _combine_body,
        grid=(_N // _BLK,),
        in_specs=[
            pl.BlockSpec((_NC, _BLK, _F), lambda i: (0, i, 0)),
            pl.BlockSpec((_NC, _BLK, _CW), lambda i: (0, i, 0)),
            pl.BlockSpec((_BLK, _F), lambda i: (i, 0)),
            pl.BlockSpec((_F, _F), lambda i: (0, 0)),
        ],
        out_specs=pl.BlockSpec((_BLK, _F), lambda i: (i, 0)),
        out_shape=jax.ShapeDtypeStruct((_N, _F), jnp.float32),
    )(seg, cnt, r, Wl)


def _combine_pool_body(s_ref, c_ref, r_ref, w_ref, p_ref):
    cl = c_ref[...]
    cnt = jnp.maximum(cl[0, :, 0:1] + cl[1, :, 0:1], 1.0)
    agg = (s_ref[0] + s_ref[1]) / cnt
    hnew = jnp.maximum(_mm_t(agg, w_ref[...]) + r_ref[...], 0.0)
    psum = jnp.sum(hnew, axis=0, keepdims=True)

    @pl.when(pl.program_id(0) == 0)
    def _():
        p_ref[...] = jnp.zeros_like(p_ref)

    p_ref[...] += psum


def _tc_combine_pool(seg, cnt, r, Wl):
    return pl.pallas_call(
        _combine_pool_body,
        grid=(_N // _BLK,),
        in_specs=[
            pl.BlockSpec((_NC, _BLK, _F), lambda i: (0, i, 0)),
            pl.BlockSpec((_NC, _BLK, _CW), lambda i: (0, i, 0)),
            pl.BlockSpec((_BLK, _F), lambda i: (i, 0)),
            pl.BlockSpec((_F, _F), lambda i: (0, 0)),
        ],
        out_specs=pl.BlockSpec((1, _F), lambda i: (0, 0)),
        out_shape=jax.ShapeDtypeStruct((1, _F), jnp.float32),
    )(seg, cnt, r, Wl)


def _decoder_body(p_ref, wg, bg, w1, b1, w2, b2, w3, b3, w4, b4, o_ref):
    g = p_ref[...] * (1.0 / _N)
    z = jax.nn.sigmoid(_mm_t(g, wg[...]) + bg[...])
    h = jnp.maximum(_mm_t(z, w1[...]) + b1[...], 0.0)
    h = jnp.maximum(_mm_t(h, w2[...]) + b2[...], 0.0)
    h = jnp.maximum(_mm_t(h, w3[...]) + b3[...], 0.0)
    o_ref[...] = _mm_t(h, w4[...]) + b4[...]


def _tc_decoder(p, Wg, bg, Wd1, bd1, Wd2, bd2, Wd3, bd3, Wd4, bd4):
    args = (p, Wg, bg, Wd1, bd1, Wd2, bd2, Wd3, bd3, Wd4, bd4)
    return pl.pallas_call(
        _decoder_body,
        out_shape=jax.ShapeDtypeStruct((1, Wd4.shape[0]), jnp.float32),
    )(*args)


def kernel(x, edge_index, W1l, b1l, W1r, W2l, b2l, W2r, W3l, b3l, W3r,
           Wg, bg, Wd1, bd1, Wd2, bd2, Wd3, bd3, Wd4, bd4):
    src = edge_index[0]
    dst = edge_index[1]
    e = src.shape[0]
    srcs = jnp.concatenate(
        [src, jnp.zeros((_EPAD - e,), jnp.int32)]).reshape(_NW, _NCH, _CK)
    dsts = jnp.concatenate(
        [dst, jnp.full((_EPAD - e,), _N, jnp.int32)]).reshape(_NW, _NCH, _CK)
    zeros_f = jnp.zeros((_CK, _F), jnp.float32)
    zeros_cw = jnp.zeros((_CK, _CW), jnp.float32)
    ones_cw = jnp.ones((_CK, _CW), jnp.float32)

    cnt = _sc_cnt(dsts, zeros_cw, ones_cw)
    seg1 = _sc_segsum(x, srcs, dsts, zeros_f)
    r1 = _tc_root(x, W1r, b1l.reshape(1, _F))
    h1 = _tc_combine(seg1, cnt, r1, W1l)

    seg2 = _sc_segsum(h1, srcs, dsts, zeros_f)
    r2 = _tc_root(h1, W2r, b2l.reshape(1, _F))
    h2 = _tc_combine(seg2, cnt, r2, W2l)

    seg3 = _sc_segsum(h2, srcs, dsts, zeros_f)
    r3 = _tc_root(h2, W3r, b3l.reshape(1, _F))
    p = _tc_combine_pool(seg3, cnt, r3, W3l)

    d = _tc_decoder(p, Wg, bg.reshape(1, -1), Wd1, bd1.reshape(1, -1),
                    Wd2, bd2.reshape(1, -1), Wd3, bd3.reshape(1, -1),
                    Wd4, bd4.reshape(1, -1))
    return d.reshape(21, _F)


# speedup vs baseline: 1.2503x; 1.2503x over previous
import functools

import jax
import jax.numpy as jnp
from jax import lax
from jax.experimental import pallas as pl
from jax.experimental.pallas import tpu as pltpu
from jax.experimental.pallas import tpu_sc as plsc

_N = 10000
_F = 128
_NC = 2
_NS = 16
_NW = _NC * _NS
_CK = 64
_NCH = 160
_NCHH = 80
_NBIG = 240
_NSML = 80
_PBIG = (48, 48, 48, 48, 48)
_PSML = (40, 40)
_PMAX = 48
_ACC = 10240
_RPT = _ACC // _NS
_EPAD = _NW * _NCH * _CK

_mesh = plsc.VectorSubcoreMesh(core_axis_name="c", subcore_axis_name="s")


_CW = 128


@functools.partial(
    pl.kernel,
    out_type=jax.ShapeDtypeStruct((_NC, _ACC, _CW), jnp.float32),
    mesh=_mesh,
    scratch_types=[
        pltpu.VMEM((_NCH, _CK), jnp.int32),
        pltpu.VMEM((_CK, _CW), jnp.float32),
        pltpu.VMEM_SHARED((_ACC, _CW), jnp.float32),
    ],
)
def _sc_cnt(dsts, zeros_f, ones_f, cnt_out, dst_v, buf_v, cacc):
    c = lax.axis_index("c")
    s = lax.axis_index("s")
    w = c * _NS + s
    pltpu.sync_copy(dsts.at[w], dst_v)
    pltpu.sync_copy(zeros_f, buf_v)

    @pl.loop(0, _ACC // (_NS * _CK))
    def _(zb):
        base = (s * (_ACC // (_NS * _CK)) + zb) * _CK
        pltpu.sync_copy(buf_v, cacc.at[pl.ds(base, _CK)])

    pltpu.sync_copy(ones_f, buf_v)
    plsc.subcore_barrier()

    @pl.loop(0, _NCH)
    def _(j):
        pltpu.sync_copy(buf_v, cacc.at[dst_v.at[j]], add=True)

    plsc.subcore_barrier()
    pltpu.sync_copy(cacc.at[pl.ds(s * _RPT, _RPT)],
                    cnt_out.at[c, pl.ds(s * _RPT, _RPT)])


def _make_segsum(big_core):
    @functools.partial(
        pl.kernel,
        out_type=jax.ShapeDtypeStruct((_NC, _ACC, _F), jnp.float32),
        mesh=_mesh,
        scratch_types=[
            pltpu.VMEM((_PMAX, _CK), jnp.int32),
            pltpu.VMEM((_PMAX, _CK), jnp.int32),
            pltpu.VMEM((_CK, _F), jnp.float32),
            pltpu.VMEM((_CK, _F), jnp.float32),
            pltpu.VMEM_SHARED((_ACC, _F), jnp.float32),
            pltpu.SemaphoreType.DMA,
            pltpu.SemaphoreType.DMA,
        ],
    )
    def _sc_segsum(h, srcs_b, dsts_b, srcs_s, dsts_s, zeros_f, seg_out,
                   src_v, dst_v, rows_a, rows_b, acc, sema, semb):
        c = lax.axis_index("c")
        s = lax.axis_index("s")
        pltpu.sync_copy(zeros_f, rows_a)

        @pl.loop(0, _ACC // (_NS * _CK))
        def _(zb):
            base = (s * (_ACC // (_NS * _CK)) + zb) * _CK
            pltpu.sync_copy(rows_a, acc.at[pl.ds(base, _CK)])

        plsc.subcore_barrier()

        def run(parts, srcs, dsts):
            off = 0
            for plen in parts:
                pltpu.sync_copy(srcs.at[s, pl.ds(off, plen)],
                                src_v.at[pl.ds(0, plen)])
                pltpu.sync_copy(dsts.at[s, pl.ds(off, plen)],
                                dst_v.at[pl.ds(0, plen)])

                @pl.loop(0, plen, step=2)
                def _(j):
                    da = pltpu.async_copy(h.at[src_v.at[j]], rows_a, sema)
                    db = pltpu.async_copy(h.at[src_v.at[j + 1]], rows_b, semb)
                    da.wait()
                    pltpu.sync_copy(rows_a, acc.at[dst_v.at[j]], add=True)
                    db.wait()
                    pltpu.sync_copy(rows_b, acc.at[dst_v.at[j + 1]], add=True)

                off += plen

        @pl.when(c == big_core)
        def _():
            run(_PBIG, srcs_b, dsts_b)

        @pl.when(c != big_core)
        def _():
            run(_PSML, srcs_s, dsts_s)

        plsc.subcore_barrier()
        pltpu.sync_copy(acc.at[pl.ds(s * _RPT, _RPT)],
                        seg_out.at[c, pl.ds(s * _RPT, _RPT)])

    return _sc_segsum


_BIG_CORE = 0
_sc_segsum_asym = _make_segsum(_BIG_CORE)


_BLK = 2000


def _mm_t(a, w):
    return lax.dot_general(a, w, (((1,), (1,)), ((), ())),
                           preferred_element_type=jnp.float32)


def _root_body(h_ref, w_ref, b_ref, o_ref):
    o_ref[...] = _mm_t(h_ref[...], w_ref[...]) + b_ref[...]


def _tc_root(h, Wr, bl2d):
    return pl.pallas_call(
        _root_body,
        grid=(_N // _BLK,),
        in_specs=[
            pl.BlockSpec((_BLK, _F), lambda i: (i, 0)),
            pl.BlockSpec((_F, _F), lambda i: (0, 0)),
            pl.BlockSpec((1, _F), lambda i: (0, 0)),
        ],
        out_specs=pl.BlockSpec((_BLK, _F), lambda i: (i, 0)),
        out_shape=jax.ShapeDtypeStruct((_N, _F), jnp.float32),
    )(h, Wr, bl2d)


def _combine_body(s_ref, c_ref, r_ref, w_ref, o_ref):
    cl = c_ref[...]
    cnt = jnp.maximum(cl[0, :, 0:1] + cl[1, :, 0:1], 1.0)
    agg = (s_ref[0] + s_ref[1]) / cnt
    o_ref[...] = jnp.maximum(_mm_t(agg, w_ref[...]) + r_ref[...], 0.0)


def _tc_combine(seg, cnt, r, Wl):
    return pl.pallas_call(
        _combine_body,
        grid=(_N // _BLK,),
        in_specs=[
            pl.BlockSpec((_NC, _BLK, _F), lambda i: (0, i, 0)),
            pl.BlockSpec((_NC, _BLK, _CW), lambda i: (0, i, 0)),
            pl.BlockSpec((_BLK, _F), lambda i: (i, 0)),
            pl.BlockSpec((_F, _F), lambda i: (0, 0)),
        ],
        out_specs=pl.BlockSpec((_BLK, _F), lambda i: (i, 0)),
        out_shape=jax.ShapeDtypeStruct((_N, _F), jnp.float32),
    )(seg, cnt, r, Wl)


def _combine_pool_body(s_ref, c_ref, r_ref, w_ref, p_ref):
    cl = c_ref[...]
    cnt = jnp.maximum(cl[0, :, 0:1] + cl[1, :, 0:1], 1.0)
    agg = (s_ref[0] + s_ref[1]) / cnt
    hnew = jnp.maximum(_mm_t(agg, w_ref[...]) + r_ref[...], 0.0)
    psum = jnp.sum(hnew, axis=0, keepdims=True)

    @pl.when(pl.program_id(0) == 0)
    def _():
        p_ref[...] = jnp.zeros_like(p_ref)

    p_ref[...] += psum


def _tc_combine_pool(seg, cnt, r, Wl):
    return pl.pallas_call(
        _combine_pool_body,
        grid=(_N // _BLK,),
        in_specs=[
            pl.BlockSpec((_NC, _BLK, _F), lambda i: (0, i, 0)),
            pl.BlockSpec((_NC, _BLK, _CW), lambda i: (0, i, 0)),
            pl.BlockSpec((_BLK, _F), lambda i: (i, 0)),
            pl.BlockSpec((_F, _F), lambda i: (0, 0)),
        ],
        out_specs=pl.BlockSpec((1, _F), lambda i: (0, 0)),
        out_shape=jax.ShapeDtypeStruct((1, _F), jnp.float32),
    )(seg, cnt, r, Wl)


def _decoder_body(p_ref, wg, bg, w1, b1, w2, b2, w3, b3, w4, b4, o_ref):
    g = p_ref[...] * (1.0 / _N)
    z = jax.nn.sigmoid(_mm_t(g, wg[...]) + bg[...])
    h = jnp.maximum(_mm_t(z, w1[...]) + b1[...], 0.0)
    h = jnp.maximum(_mm_t(h, w2[...]) + b2[...], 0.0)
    h = jnp.maximum(_mm_t(h, w3[...]) + b3[...], 0.0)
    o_ref[...] = _mm_t(h, w4[...]) + b4[...]


def _tc_decoder(p, Wg, bg, Wd1, bd1, Wd2, bd2, Wd3, bd3, Wd4, bd4):
    args = (p, Wg, bg, Wd1, bd1, Wd2, bd2, Wd3, bd3, Wd4, bd4)
    return pl.pallas_call(
        _decoder_body,
        out_shape=jax.ShapeDtypeStruct((1, Wd4.shape[0]), jnp.float32),
    )(*args)


def kernel(x, edge_index, W1l, b1l, W1r, W2l, b2l, W2r, W3l, b3l, W3r,
           Wg, bg, Wd1, bd1, Wd2, bd2, Wd3, bd3, Wd4, bd4):
    src = edge_index[0]
    dst = edge_index[1]
    e = src.shape[0]
    srcs = jnp.concatenate(
        [src, jnp.zeros((_EPAD - e,), jnp.int32)]).reshape(_NW, _NCH, _CK)
    dsts = jnp.concatenate(
        [dst, jnp.full((_EPAD - e,), _N, jnp.int32)]).reshape(_NW, _NCH, _CK)
    eb = _NS * _NBIG * _CK
    es = _NS * _NSML * _CK
    srcs_b = src[:eb].reshape(_NS, _NBIG, _CK)
    dsts_b = dst[:eb].reshape(_NS, _NBIG, _CK)
    srcs_s = jnp.concatenate(
        [src[eb:], jnp.zeros((eb + es - e,), jnp.int32)]).reshape(_NS, _NSML, _CK)
    dsts_s = jnp.concatenate(
        [dst[eb:], jnp.full((eb + es - e,), _N, jnp.int32)]).reshape(_NS, _NSML, _CK)
    zeros_f = jnp.zeros((_CK, _F), jnp.float32)
    zeros_cw = jnp.zeros((_CK, _CW), jnp.float32)
    ones_cw = jnp.ones((_CK, _CW), jnp.float32)

    cnt = _sc_cnt(dsts, zeros_cw, ones_cw)
    seg1 = _sc_segsum_asym(x, srcs_b, dsts_b, srcs_s, dsts_s, zeros_f)
    r1 = _tc_root(x, W1r, b1l.reshape(1, _F))
    h1 = _tc_combine(seg1, cnt, r1, W1l)

    seg2 = _sc_segsum_asym(h1, srcs_b, dsts_b, srcs_s, dsts_s, zeros_f)
    r2 = _tc_root(h1, W2r, b2l.reshape(1, _F))
    h2 = _tc_combine(seg2, cnt, r2, W2l)

    seg3 = _sc_segsum_asym(h2, srcs_b, dsts_b, srcs_s, dsts_s, zeros_f)
    r3 = _tc_root(h2, W3r, b3l.reshape(1, _F))
    p = _tc_combine_pool(seg3, cnt, r3, W3l)

    d = _tc_decoder(p, Wg, bg.reshape(1, -1), Wd1, bd1.reshape(1, -1),
                    Wd2, bd2.reshape(1, -1), Wd3, bd3.reshape(1, -1),
                    Wd4, bd4.reshape(1, -1))
    return d.reshape(21, _F)


# speedup vs baseline: 1.3249x; 1.0596x over previous
import functools

import jax
import jax.numpy as jnp
from jax import lax
from jax.experimental import pallas as pl
from jax.experimental.pallas import tpu as pltpu
from jax.experimental.pallas import tpu_sc as plsc

_N = 10000
_F = 128
_NC = 2
_NS = 16
_NW = _NC * _NS
_CK = 64
_NCH = 160
_NCHH = 80
_NBIG = 240
_NSML = 80
_PBIG = (48, 48, 48, 48, 48)
_PSML = (40, 40)
_PMAX = 48
_ACC = 10240
_RPT = _ACC // _NS
_EPAD = _NW * _NCH * _CK

_mesh = plsc.VectorSubcoreMesh(core_axis_name="c", subcore_axis_name="s")


_CW = 128


@functools.partial(
    pl.kernel,
    out_type=jax.ShapeDtypeStruct((_NC, _ACC, _CW), jnp.float32),
    mesh=_mesh,
    scratch_types=[
        pltpu.VMEM((_NCH, _CK), jnp.int32),
        pltpu.VMEM((_CK, _CW), jnp.float32),
        pltpu.VMEM_SHARED((_ACC, _CW), jnp.float32),
    ],
)
def _sc_cnt(dsts, zeros_f, ones_f, cnt_out, dst_v, buf_v, cacc):
    c = lax.axis_index("c")
    s = lax.axis_index("s")
    w = c * _NS + s
    pltpu.sync_copy(dsts.at[w], dst_v)
    pltpu.sync_copy(zeros_f, buf_v)

    @pl.loop(0, _ACC // (_NS * _CK))
    def _(zb):
        base = (s * (_ACC // (_NS * _CK)) + zb) * _CK
        pltpu.sync_copy(buf_v, cacc.at[pl.ds(base, _CK)])

    pltpu.sync_copy(ones_f, buf_v)
    plsc.subcore_barrier()

    @pl.loop(0, _NCH)
    def _(j):
        pltpu.sync_copy(buf_v, cacc.at[dst_v.at[j]], add=True)

    plsc.subcore_barrier()
    pltpu.sync_copy(cacc.at[pl.ds(s * _RPT, _RPT)],
                    cnt_out.at[c, pl.ds(s * _RPT, _RPT)])


def _make_segsum(big_core):
    @functools.partial(
        pl.kernel,
        out_type=jax.ShapeDtypeStruct((_NC, _ACC, _F), jnp.float32),
        mesh=_mesh,
        scratch_types=[
            pltpu.VMEM((_PMAX, _CK), jnp.int32),
            pltpu.VMEM((_PMAX, _CK), jnp.int32),
            pltpu.VMEM((_CK, _F), jnp.float32),
            pltpu.VMEM((_CK, _F), jnp.float32),
            pltpu.VMEM_SHARED((_ACC, _F), jnp.float32),
            pltpu.SemaphoreType.DMA,
            pltpu.SemaphoreType.DMA,
        ],
    )
    def _sc_segsum(h, srcs_b, dsts_b, srcs_s, dsts_s, zeros_f, seg_out,
                   src_v, dst_v, rows_a, rows_b, acc, sema, semb):
        c = lax.axis_index("c")
        s = lax.axis_index("s")
        pltpu.sync_copy(zeros_f, rows_a)

        @pl.loop(0, _ACC // (_NS * _CK))
        def _(zb):
            base = (s * (_ACC // (_NS * _CK)) + zb) * _CK
            pltpu.sync_copy(rows_a, acc.at[pl.ds(base, _CK)])

        plsc.subcore_barrier()

        def run(parts, srcs, dsts):
            off = 0
            for plen in parts:
                pltpu.sync_copy(srcs.at[s, pl.ds(off, plen)],
                                src_v.at[pl.ds(0, plen)])
                pltpu.sync_copy(dsts.at[s, pl.ds(off, plen)],
                                dst_v.at[pl.ds(0, plen)])

                @pl.loop(0, plen, step=2)
                def _(j):
                    da = pltpu.async_copy(h.at[src_v.at[j]], rows_a, sema)
                    db = pltpu.async_copy(h.at[src_v.at[j + 1]], rows_b, semb)
                    da.wait()
                    pltpu.sync_copy(rows_a, acc.at[dst_v.at[j]], add=True)
                    db.wait()
                    pltpu.sync_copy(rows_b, acc.at[dst_v.at[j + 1]], add=True)

                off += plen

        @pl.when(c == big_core)
        def _():
            run(_PBIG, srcs_b, dsts_b)

        @pl.when(c != big_core)
        def _():
            run(_PSML, srcs_s, dsts_s)

        plsc.subcore_barrier()
        pltpu.sync_copy(acc.at[pl.ds(s * _RPT, _RPT)],
                        seg_out.at[c, pl.ds(s * _RPT, _RPT)])

    return _sc_segsum


_BIG_CORE = 1
_sc_segsum_asym = _make_segsum(_BIG_CORE)


_BLK = 2000


def _mm_t(a, w):
    return lax.dot_general(a, w, (((1,), (1,)), ((), ())),
                           preferred_element_type=jnp.float32)


def _root_body(h_ref, w_ref, b_ref, o_ref):
    o_ref[...] = _mm_t(h_ref[...], w_ref[...]) + b_ref[...]


def _tc_root(h, Wr, bl2d):
    return pl.pallas_call(
        _root_body,
        grid=(_N // _BLK,),
        in_specs=[
            pl.BlockSpec((_BLK, _F), lambda i: (i, 0)),
            pl.BlockSpec((_F, _F), lambda i: (0, 0)),
            pl.BlockSpec((1, _F), lambda i: (0, 0)),
        ],
        out_specs=pl.BlockSpec((_BLK, _F), lambda i: (i, 0)),
        out_shape=jax.ShapeDtypeStruct((_N, _F), jnp.float32),
    )(h, Wr, bl2d)


def _combine_body(s_ref, c_ref, r_ref, w_ref, o_ref):
    cl = c_ref[...]
    cnt = jnp.maximum(cl[0, :, 0:1] + cl[1, :, 0:1], 1.0)
    agg = (s_ref[0] + s_ref[1]) / cnt
    o_ref[...] = jnp.maximum(_mm_t(agg, w_ref[...]) + r_ref[...], 0.0)


def _tc_combine(seg, cnt, r, Wl):
    return pl.pallas_call(
        _combine_body,
        grid=(_N // _BLK,),
        in_specs=[
            pl.BlockSpec((_NC, _BLK, _F), lambda i: (0, i, 0)),
            pl.BlockSpec((_NC, _BLK, _CW), lambda i: (0, i, 0)),
            pl.BlockSpec((_BLK, _F), lambda i: (i, 0)),
            pl.BlockSpec((_F, _F), lambda i: (0, 0)),
        ],
        out_specs=pl.BlockSpec((_BLK, _F), lambda i: (i, 0)),
        out_shape=jax.ShapeDtypeStruct((_N, _F), jnp.float32),
    )(seg, cnt, r, Wl)


def _combine_pool_body(s_ref, c_ref, r_ref, w_ref, p_ref):
    cl = c_ref[...]
    cnt = jnp.maximum(cl[0, :, 0:1] + cl[1, :, 0:1], 1.0)
    agg = (s_ref[0] + s_ref[1]) / cnt
    hnew = jnp.maximum(_mm_t(agg, w_ref[...]) + r_ref[...], 0.0)
    psum = jnp.sum(hnew, axis=0, keepdims=True)

    @pl.when(pl.program_id(0) == 0)
    def _():
        p_ref[...] = jnp.zeros_like(p_ref)

    p_ref[...] += psum


def _tc_combine_pool(seg, cnt, r, Wl):
    return pl.pallas_call(
        _combine_pool_body,
        grid=(_N // _BLK,),
        in_specs=[
            pl.BlockSpec((_NC, _BLK, _F), lambda i: (0, i, 0)),
            pl.BlockSpec((_NC, _BLK, _CW), lambda i: (0, i, 0)),
            pl.BlockSpec((_BLK, _F), lambda i: (i, 0)),
            pl.BlockSpec((_F, _F), lambda i: (0, 0)),
        ],
        out_specs=pl.BlockSpec((1, _F), lambda i: (0, 0)),
        out_shape=jax.ShapeDtypeStruct((1, _F), jnp.float32),
    )(seg, cnt, r, Wl)


def _decoder_body(p_ref, wg, bg, w1, b1, w2, b2, w3, b3, w4, b4, o_ref):
    g = p_ref[...] * (1.0 / _N)
    z = jax.nn.sigmoid(_mm_t(g, wg[...]) + bg[...])
    h = jnp.maximum(_mm_t(z, w1[...]) + b1[...], 0.0)
    h = jnp.maximum(_mm_t(h, w2[...]) + b2[...], 0.0)
    h = jnp.maximum(_mm_t(h, w3[...]) + b3[...], 0.0)
    o_ref[...] = _mm_t(h, w4[...]) + b4[...]


def _tc_decoder(p, Wg, bg, Wd1, bd1, Wd2, bd2, Wd3, bd3, Wd4, bd4):
    args = (p, Wg, bg, Wd1, bd1, Wd2, bd2, Wd3, bd3, Wd4, bd4)
    return pl.pallas_call(
        _decoder_body,
        out_shape=jax.ShapeDtypeStruct((1, Wd4.shape[0]), jnp.float32),
    )(*args)


def kernel(x, edge_index, W1l, b1l, W1r, W2l, b2l, W2r, W3l, b3l, W3r,
           Wg, bg, Wd1, bd1, Wd2, bd2, Wd3, bd3, Wd4, bd4):
    src = edge_index[0]
    dst = edge_index[1]
    e = src.shape[0]
    srcs = jnp.concatenate(
        [src, jnp.zeros((_EPAD - e,), jnp.int32)]).reshape(_NW, _NCH, _CK)
    dsts = jnp.concatenate(
        [dst, jnp.full((_EPAD - e,), _N, jnp.int32)]).reshape(_NW, _NCH, _CK)
    eb = _NS * _NBIG * _CK
    es = _NS * _NSML * _CK
    srcs_b = src[:eb].reshape(_NS, _NBIG, _CK)
    dsts_b = dst[:eb].reshape(_NS, _NBIG, _CK)
    srcs_s = jnp.concatenate(
        [src[eb:], jnp.zeros((eb + es - e,), jnp.int32)]).reshape(_NS, _NSML, _CK)
    dsts_s = jnp.concatenate(
        [dst[eb:], jnp.full((eb + es - e,), _N, jnp.int32)]).reshape(_NS, _NSML, _CK)
    zeros_f = jnp.zeros((_CK, _F), jnp.float32)
    zeros_cw = jnp.zeros((_CK, _CW), jnp.float32)
    ones_cw = jnp.ones((_CK, _CW), jnp.float32)

    cnt = _sc_cnt(dsts, zeros_cw, ones_cw)
    seg1 = _sc_segsum_asym(x, srcs_b, dsts_b, srcs_s, dsts_s, zeros_f)
    r1 = _tc_root(x, W1r, b1l.reshape(1, _F))
    h1 = _tc_combine(seg1, cnt, r1, W1l)

    seg2 = _sc_segsum_asym(h1, srcs_b, dsts_b, srcs_s, dsts_s, zeros_f)
    r2 = _tc_root(h1, W2r, b2l.reshape(1, _F))
    h2 = _tc_combine(seg2, cnt, r2, W2l)

    seg3 = _sc_segsum_asym(h2, srcs_b, dsts_b, srcs_s, dsts_s, zeros_f)
    r3 = _tc_root(h2, W3r, b3l.reshape(1, _F))
    p = _tc_combine_pool(seg3, cnt, r3, W3l)

    d = _tc_decoder(p, Wg, bg.reshape(1, -1), Wd1, bd1.reshape(1, -1),
                    Wd2, bd2.reshape(1, -1), Wd3, bd3.reshape(1, -1),
                    Wd4, bd4.reshape(1, -1))
    return d.reshape(21, _F)
